# Initial kernel scaffold; baseline (speedup 1.0000x reference)
import jax
import jax.numpy as jnp
from jax.experimental import pallas as pl


def kernel(inputs, gate_w, w1, w2, w3):
    raise NotImplementedError("write your pallas kernel here")


# speedup vs baseline: 1.3554x; 1.3554x over previous
import jax
import jax.numpy as jnp
from jax.experimental import pallas as pl
from jax.experimental.pallas import tpu as pltpu

E = 8
TOP_K = 2
D_MODEL = 1024
D_FF = 2048
T = 2048
BT = 256
BFF = 1024
NFF = D_FF // BFF


def _moe_block(x_ref, gate_ref, w1_ref, w3_ref, w2_ref, out_ref):
    e = pl.program_id(0)
    f = pl.program_id(1)

    def body(i, _):
        x = x_ref[pl.ds(i * BT, BT), :]
        logits = jnp.dot(x, gate_ref[...], preferred_element_type=jnp.float32)
        m1 = jnp.max(logits, axis=1, keepdims=True)
        i1 = jnp.argmax(logits, axis=1)
        col = jax.lax.broadcasted_iota(jnp.int32, logits.shape, 1)
        masked = jnp.where(col == i1[:, None], -jnp.inf, logits)
        m2 = jnp.max(masked, axis=1, keepdims=True)
        i2 = jnp.argmax(masked, axis=1)
        e2 = jnp.exp(m2 - m1)
        denom = 1.0 + e2
        w_1 = (1.0 / denom)[:, 0]
        w_2 = (e2 / denom)[:, 0]
        coef = jnp.where(i1 == e, w_1, 0.0) + jnp.where(i2 == e, w_2, 0.0)

        h = jax.nn.silu(jnp.dot(x, w1_ref[0], preferred_element_type=jnp.float32))
        h = h * jnp.dot(x, w3_ref[0], preferred_element_type=jnp.float32)
        eo = jnp.dot(h, w2_ref[0], preferred_element_type=jnp.float32)
        contrib = coef[:, None] * eo

        @pl.when((e == 0) & (f == 0))
        def _():
            out_ref[pl.ds(i * BT, BT), :] = contrib

        @pl.when((e > 0) | (f > 0))
        def _():
            out_ref[pl.ds(i * BT, BT), :] += contrib

        return 0

    jax.lax.fori_loop(0, T // BT, body, 0)


def kernel(inputs, gate_w, w1, w2, w3):
    return pl.pallas_call(
        _moe_block,
        grid=(E, NFF),
        in_specs=[
            pl.BlockSpec((T, D_MODEL), lambda e, f: (0, 0)),
            pl.BlockSpec((D_MODEL, E), lambda e, f: (0, 0)),
            pl.BlockSpec((1, D_MODEL, BFF), lambda e, f: (e, 0, f)),
            pl.BlockSpec((1, D_MODEL, BFF), lambda e, f: (e, 0, f)),
            pl.BlockSpec((1, BFF, D_MODEL), lambda e, f: (e, f, 0)),
        ],
        out_specs=pl.BlockSpec((T, D_MODEL), lambda e, f: (0, 0)),
        out_shape=jax.ShapeDtypeStruct((T, D_MODEL), jnp.float32),
        compiler_params=pltpu.CompilerParams(
            dimension_semantics=("arbitrary", "arbitrary"),
        ),
    )(inputs, gate_w, w1, w3, w2)
